# Initial kernel scaffold; baseline (speedup 1.0000x reference)
import jax
import jax.numpy as jnp
from jax.experimental import pallas as pl


def kernel(x):
    raise NotImplementedError("write your pallas kernel here")


# speedup vs baseline: 1.7776x; 1.7776x over previous
import functools

import jax
import jax.numpy as jnp
from jax import lax
from jax.experimental import pallas as pl
from jax.experimental.pallas import tpu as pltpu
from jax.experimental.pallas import tpu_sc as plsc

_K = 64
_NROWS = 128
_N = 32768
_LANES = 16
_NWORKERS = 32
_ROWS_PER_W = _NROWS // _NWORKERS
_CHUNKS = _N // _LANES
_HSTRIDE = 272
_HSIZE = _LANES * _HSTRIDE


def _f32_key(v):
    b = plsc.bitcast(v, jnp.int32)
    return b ^ (jnp.right_shift(b, 31) & jnp.int32(0x7FFFFFFF))


def _clear_hist(hist_v):
    def body(i, _):
        hist_v[pl.ds(i * _LANES, _LANES)] = jnp.zeros((_LANES,), jnp.int32)
        return jnp.int32(0)

    lax.fori_loop(0, _HSIZE // _LANES, body, jnp.int32(0))


def _suffix_and_bucket(hist_v, suf_v, lanes, r):
    def tot_body(g, _):
        def add_body(l, acc):
            return acc + hist_v[pl.ds(l * _HSTRIDE + g * _LANES, _LANES)]

        acc = lax.fori_loop(0, _LANES, add_body, jnp.zeros((_LANES,), jnp.int32))
        suf_v[pl.ds(g * _LANES, _LANES)] = acc
        return jnp.int32(0)

    lax.fori_loop(0, 16, tot_body, jnp.int32(0))

    def suf_body(g2, carry):
        g = 15 - g2
        v = suf_v[pl.ds(g * _LANES, _LANES)]
        s = lax.rev(jnp.cumsum(lax.rev(v, (0,))), (0,)) + carry
        suf_v[pl.ds(g * _LANES, _LANES)] = s
        return (carry + jnp.sum(v)).astype(jnp.int32)

    lax.fori_loop(0, 16, suf_body, jnp.int32(0))

    def find_body(g, b):
        bids = g * _LANES + lanes
        sv = suf_v[pl.ds(g * _LANES, _LANES)]
        cand = jnp.where(sv >= r, bids, -1)
        return jnp.maximum(b, jnp.max(cand)).astype(jnp.int32)

    return lax.fori_loop(0, 16, find_body, jnp.int32(-1))


def _sc_body(x_hbm, outk_hbm, outi_hbm, row_v, ck_v, ci_v, hist_v, suf_v,
             wk_v, wi_v):
    wid = lax.axis_index("s") * 2 + lax.axis_index("c")
    lanes = lax.iota(jnp.int32, _LANES)
    lane_off = lanes * _HSTRIDE

    def row_body(j, _):
        row = (wid * _ROWS_PER_W + j).astype(jnp.int32)
        pltpu.sync_copy(x_hbm.at[row], row_v)

        _clear_hist(hist_v)

        def hist_body(c, _):
            key = _f32_key(row_v[pl.ds(c * _LANES, _LANES)])
            dig = jnp.right_shift(key, 24) + 128
            addr = lane_off + dig
            cnt = plsc.load_gather(hist_v, [addr])
            plsc.store_scatter(hist_v, [addr], cnt + 1)
            return jnp.int32(0)

        lax.fori_loop(0, _CHUNKS, hist_body, jnp.int32(0))

        b1 = _suffix_and_bucket(hist_v, suf_v, lanes, jnp.int32(_K))

        def p2_body(c, carry):
            wn, cn = carry
            key = _f32_key(row_v[pl.ds(c * _LANES, _LANES)])
            dig = jnp.right_shift(key, 24) + 128
            oi = c * _LANES + lanes
            gt = dig > b1
            eq = dig == b1
            gti = gt.astype(jnp.int32)
            eqi = eq.astype(jnp.int32)
            posw = wn + jnp.cumsum(gti) - gti
            plsc.store_scatter(wk_v, [posw], key, mask=gt)
            plsc.store_scatter(wi_v, [posw], oi, mask=gt)
            posc = cn + jnp.cumsum(eqi) - eqi
            plsc.store_scatter(ck_v, [posc], key, mask=eq)
            plsc.store_scatter(ci_v, [posc], oi, mask=eq)
            return (wn + plsc.all_reduce_population_count(gt),
                    cn + plsc.all_reduce_population_count(eq))

        zero = jnp.zeros((_LANES,), jnp.int32)
        wn_s, cn_s = lax.fori_loop(0, _CHUNKS, p2_body, (zero, zero))
        wn = jnp.max(wn_s).astype(jnp.int32)
        m = jnp.max(cn_s).astype(jnp.int32)

        def level_body(lvl, carry):
            m, wn = carry
            r = jnp.int32(_K) - wn
            shift = jnp.full((_LANES,), 16 - lvl * 8, jnp.int32)
            nch = (m + _LANES - 1) // _LANES
            _clear_hist(hist_v)

            def h_body(c, _):
                key = ck_v[pl.ds(c * _LANES, _LANES)]
                valid = (c * _LANES + lanes) < m
                dig = jnp.where(
                    valid, lax.shift_right_logical(key, shift) & 0xFF, 260)
                addr = lane_off + dig
                cnt = plsc.load_gather(hist_v, [addr])
                plsc.store_scatter(hist_v, [addr], cnt + 1)
                return jnp.int32(0)

            lax.fori_loop(0, nch, h_body, jnp.int32(0))

            bl = _suffix_and_bucket(hist_v, suf_v, lanes, r)

            def c_body(c, carry2):
                wn2, cn2 = carry2
                key = ck_v[pl.ds(c * _LANES, _LANES)]
                oi = ci_v[pl.ds(c * _LANES, _LANES)]
                valid = (c * _LANES + lanes) < m
                dig = jnp.where(
                    valid, lax.shift_right_logical(key, shift) & 0xFF, -1)
                gt = dig > bl
                eq = dig == bl
                gti = gt.astype(jnp.int32)
                eqi = eq.astype(jnp.int32)
                posw = wn2 + jnp.cumsum(gti) - gti
                plsc.store_scatter(wk_v, [posw], key, mask=gt)
                plsc.store_scatter(wi_v, [posw], oi, mask=gt)
                posc = cn2 + jnp.cumsum(eqi) - eqi
                plsc.store_scatter(ck_v, [posc], key, mask=eq)
                plsc.store_scatter(ci_v, [posc], oi, mask=eq)
                return ((wn2 + jnp.sum(gti)).astype(jnp.int32),
                        (cn2 + jnp.sum(eqi)).astype(jnp.int32))

            wn_new, m_new = lax.fori_loop(0, nch, c_body, (wn, jnp.int32(0)))
            return (m_new, wn_new)

        m, wn = lax.fori_loop(0, 3, level_body, (m, wn))

        r = jnp.int32(_K) - wn
        nfin = (r + _LANES - 1) // _LANES

        def f_body(c, _):
            key = ck_v[pl.ds(c * _LANES, _LANES)]
            oi = ci_v[pl.ds(c * _LANES, _LANES)]
            p = c * _LANES + lanes
            sel = p < r
            plsc.store_scatter(wk_v, [wn + p], key, mask=sel)
            plsc.store_scatter(wi_v, [wn + p], oi, mask=sel)
            return jnp.int32(0)

        lax.fori_loop(0, nfin, f_body, jnp.int32(0))

        pltpu.sync_copy(wk_v.at[pl.ds(0, _K)], outk_hbm.at[row])
        pltpu.sync_copy(wi_v.at[pl.ds(0, _K)], outi_hbm.at[row])
        return jnp.int32(0)

    lax.fori_loop(0, _ROWS_PER_W, row_body, jnp.int32(0))


_sc_topk = functools.partial(
    pl.kernel,
    out_type=(
        jax.ShapeDtypeStruct((_NROWS, _K), jnp.int32),
        jax.ShapeDtypeStruct((_NROWS, _K), jnp.int32),
    ),
    mesh=plsc.VectorSubcoreMesh(core_axis_name="c", subcore_axis_name="s"),
    compiler_params=pltpu.CompilerParams(
        needs_layout_passes=False, use_tc_tiling_on_sc=False),
    scratch_types=[
        pltpu.VMEM((_N,), jnp.float32),
        pltpu.VMEM((_N,), jnp.int32),
        pltpu.VMEM((_N,), jnp.int32),
        pltpu.VMEM((_HSIZE,), jnp.int32),
        pltpu.VMEM((_HSTRIDE,), jnp.int32),
        pltpu.VMEM((80,), jnp.int32),
        pltpu.VMEM((80,), jnp.int32),
    ],
)(_sc_body)


def _tc_sort_body(k_ref, i_ref, vo_ref, io_ref):
    k = k_ref[...]
    ix = i_ref[...]
    lane = lax.broadcasted_iota(jnp.int32, (_NROWS, _K), 1)
    for kk in [2, 4, 8, 16, 32, 64]:
        j = kk // 2
        while j >= 1:
            lk = jnp.concatenate([k[:, j:], k[:, :j]], axis=1)
            rk = jnp.concatenate([k[:, _K - j:], k[:, :_K - j]], axis=1)
            li = jnp.concatenate([ix[:, j:], ix[:, :j]], axis=1)
            ri = jnp.concatenate([ix[:, _K - j:], ix[:, :_K - j]], axis=1)
            bitclear = (lane & j) == 0
            pk = jnp.where(bitclear, lk, rk)
            pi = jnp.where(bitclear, li, ri)
            first = (k > pk) | ((k == pk) & (ix < pi))
            forward = (lane & kk) == 0
            take = bitclear ^ first ^ (~forward)
            k = jnp.where(take, pk, k)
            ix = jnp.where(take, pi, ix)
            j //= 2
    bits = k ^ (jnp.right_shift(k, 31) & jnp.int32(0x7FFFFFFF))
    vo_ref[...] = lax.bitcast_convert_type(bits, jnp.float32)
    io_ref[...] = ix


_tc_sort = pl.pallas_call(
    _tc_sort_body,
    out_shape=(
        jax.ShapeDtypeStruct((_NROWS, _K), jnp.float32),
        jax.ShapeDtypeStruct((_NROWS, _K), jnp.int32),
    ),
)


def kernel(x):
    wk, wi = _sc_topk(x)
    return _tc_sort(wk, wi)


# speedup vs baseline: 2.2141x; 1.2455x over previous
import functools

import jax
import jax.numpy as jnp
from jax import lax
from jax.experimental import pallas as pl
from jax.experimental.pallas import tpu as pltpu
from jax.experimental.pallas import tpu_sc as plsc

_K = 64
_NROWS = 128
_N = 32768
_LANES = 16
_NWORKERS = 32
_ROWS_PER_W = _NROWS // _NWORKERS
_CHUNKS = _N // _LANES


def _f32_key(v):
    b = plsc.bitcast(v, jnp.int32)
    return b ^ (jnp.right_shift(b, 31) & jnp.int32(0x7FFFFFFF))


def _sc_body(x_hbm, outv_hbm, outi_hbm, row_a, row_b, ci_v, h1_v, h2_v,
             suf_v, wk_v, wi_v, sem_a, sem_b):
    wid = lax.axis_index("s") * 2 + lax.axis_index("c")
    lanes = lax.iota(jnp.int32, _LANES)
    ones = jnp.ones((_LANES,), jnp.int32)
    zvec = jnp.zeros((_LANES,), jnp.int32)
    l1off = lanes * 256
    l2off = lanes * 17
    base_row = wid * _ROWS_PER_W

    bufs = (row_a, row_b)
    sems = (sem_a, sem_b)
    pending = pltpu.async_copy(x_hbm.at[base_row], row_a, sem_a)
    for j in range(_ROWS_PER_W):
        cur = bufs[j % 2]
        if j + 1 < _ROWS_PER_W:
            nxt = pltpu.async_copy(
                x_hbm.at[base_row + j + 1], bufs[(j + 1) % 2],
                sems[(j + 1) % 2])
        pending.wait()

        def cl1_body(i, _):
            h1_v[pl.ds(i * _LANES, _LANES)] = zvec
            return jnp.int32(0)

        lax.fori_loop(0, 256, cl1_body, jnp.int32(0))

        def h1_body(c0, _):
            for u in range(4):
                key = _f32_key(cur[pl.ds(c0 * 64 + u * _LANES, _LANES)])
                dig = jnp.right_shift(key, 24) + 128
                plsc.addupdate_scatter(h1_v, [l1off + dig], ones)
            return jnp.int32(0)

        lax.fori_loop(0, _CHUNKS // 4, h1_body, jnp.int32(0))

        def tot_body(g, _):
            def add_body(l, acc):
                return acc + h1_v[pl.ds(l * 256 + g * _LANES, _LANES)]

            acc = lax.fori_loop(0, _LANES, add_body, zvec)
            suf_v[pl.ds(g * _LANES, _LANES)] = acc
            return jnp.int32(0)

        lax.fori_loop(0, 16, tot_body, jnp.int32(0))

        def suf_body(g2, carry):
            g = 15 - g2
            v = suf_v[pl.ds(g * _LANES, _LANES)]
            s = lax.rev(jnp.cumsum(lax.rev(v, (0,))), (0,)) + carry
            suf_v[pl.ds(g * _LANES, _LANES)] = s
            return (carry + jnp.sum(v)).astype(jnp.int32)

        lax.fori_loop(0, 16, suf_body, jnp.int32(0))

        def find_body(g, b):
            bids = g * _LANES + lanes
            sv = suf_v[pl.ds(g * _LANES, _LANES)]
            cand = jnp.where(sv >= _K, bids, -1)
            return jnp.maximum(b, jnp.max(cand)).astype(jnp.int32)

        b1 = lax.fori_loop(0, 16, find_body, jnp.int32(-1))
        thresh = jnp.left_shift(b1 - 128, 24).astype(jnp.int32)

        def p2_body(c0, cn):
            for u in range(2):
                off = c0 * 32 + u * _LANES
                key = _f32_key(cur[pl.ds(off, _LANES)])
                keep = key >= thresh
                ki = keep.astype(jnp.int32)
                pos = cn + jnp.cumsum(ki) - ki
                plsc.store_scatter(ci_v, [pos], off + lanes, mask=keep)
                cn = cn + plsc.all_reduce_population_count(keep)
            return cn

        cn = lax.fori_loop(0, _CHUNKS // 2, p2_body, zvec)
        m = jnp.max(cn).astype(jnp.int32)
        c_gt = jnp.int32(0)

        def level_body(lvl, carry):
            thresh, m, _ = carry
            s = (20 - 4 * lvl).astype(jnp.int32)

            def cl2_body(i, _):
                h2_v[pl.ds(i * _LANES, _LANES)] = zvec
                return jnp.int32(0)

            lax.fori_loop(0, 17, cl2_body, jnp.int32(0))
            nch = (m + _LANES - 1) // _LANES
            width = jnp.left_shift(jnp.int32(1), s + 4)

            def hb_body(c, _):
                idx = ci_v[pl.ds(c * _LANES, _LANES)]
                valid = (c * _LANES + lanes) < m
                kv = plsc.load_gather(cur, [idx], mask=valid)
                key = _f32_key(kv)
                d = plsc.bitcast(key - thresh, jnp.uint32)
                bnd = valid & (d < plsc.bitcast(
                    jnp.full((_LANES,), width, jnp.int32), jnp.uint32))
                dig = jnp.where(
                    bnd, jnp.right_shift(d, s.astype(jnp.uint32))
                    .astype(jnp.int32), 16)
                plsc.addupdate_scatter(h2_v, [l2off + dig], ones)
                return jnp.int32(0)

            lax.fori_loop(0, nch, hb_body, jnp.int32(0))

            tot = zvec
            for l in range(_LANES):
                tot = tot + h2_v[pl.ds(l * 17, _LANES)]
            suffix = lax.rev(jnp.cumsum(lax.rev(tot, (0,))), (0,))
            nB = jnp.sum(tot).astype(jnp.int32)
            r_l = jnp.int32(_K) - (m - nB)
            b = jnp.max(jnp.where(suffix >= r_l, lanes, -1)).astype(jnp.int32)
            sufb1 = jnp.sum(jnp.where(lanes == b + 1, suffix, 0)).astype(
                jnp.int32)
            c_gt = (m - nB) + sufb1
            thresh = (thresh + jnp.left_shift(b, s)).astype(jnp.int32)

            def cb_body(c, cn2):
                idx = ci_v[pl.ds(c * _LANES, _LANES)]
                valid = (c * _LANES + lanes) < m
                kv = plsc.load_gather(cur, [idx], mask=valid)
                key = _f32_key(kv)
                keep = valid & (key >= thresh)
                ki = keep.astype(jnp.int32)
                pos = cn2 + jnp.cumsum(ki) - ki
                plsc.store_scatter(ci_v, [pos], idx, mask=keep)
                return cn2 + plsc.all_reduce_population_count(keep)

            cn2 = lax.fori_loop(0, nch, cb_body, zvec)
            return (thresh, jnp.max(cn2).astype(jnp.int32), c_gt)

        thresh, m, c_gt = lax.fori_loop(
            0, 6, level_body, (thresh, m, c_gt))

        r_fin = jnp.int32(_K) - c_gt
        nchf = (m + _LANES - 1) // _LANES

        def f_body(c, carry):
            neq, nw = carry
            idx = ci_v[pl.ds(c * _LANES, _LANES)]
            valid = (c * _LANES + lanes) < m
            kv = plsc.load_gather(cur, [idx], mask=valid)
            key = _f32_key(kv)
            gt = valid & (key > thresh)
            eq = valid & (key == thresh)
            eqi = eq.astype(jnp.int32)
            tier = neq + jnp.cumsum(eqi) - eqi
            keep = gt | (eq & (tier < r_fin))
            ki = keep.astype(jnp.int32)
            pos = nw + jnp.cumsum(ki) - ki + (j * _K)
            plsc.store_scatter(wi_v, [pos], idx, mask=keep)
            return (neq + plsc.all_reduce_population_count(eq),
                    nw + plsc.all_reduce_population_count(keep))

        lax.fori_loop(0, nchf, f_body, (zvec, zvec))

        for c in range(_K // _LANES):
            iv = wi_v[pl.ds(j * _K + c * _LANES, _LANES)]
            wk_v[pl.ds(j * _K + c * _LANES, _LANES)] = plsc.load_gather(
                cur, [iv])

        if j + 1 < _ROWS_PER_W:
            pending = nxt

    pltpu.sync_copy(wk_v, outv_hbm.at[pl.ds(wid * (_ROWS_PER_W * _K),
                                            _ROWS_PER_W * _K)])
    pltpu.sync_copy(wi_v, outi_hbm.at[pl.ds(wid * (_ROWS_PER_W * _K),
                                            _ROWS_PER_W * _K)])


_sc_topk = functools.partial(
    pl.kernel,
    out_type=(
        jax.ShapeDtypeStruct((_NROWS * _K,), jnp.float32),
        jax.ShapeDtypeStruct((_NROWS * _K,), jnp.int32),
    ),
    mesh=plsc.VectorSubcoreMesh(core_axis_name="c", subcore_axis_name="s"),
    compiler_params=pltpu.CompilerParams(
        needs_layout_passes=False, use_tc_tiling_on_sc=False),
    scratch_types=[
        pltpu.VMEM((_N,), jnp.float32),
        pltpu.VMEM((_N,), jnp.float32),
        pltpu.VMEM((_N,), jnp.int32),
        pltpu.VMEM((16 * 256,), jnp.int32),
        pltpu.VMEM((16 * 17,), jnp.int32),
        pltpu.VMEM((256,), jnp.int32),
        pltpu.VMEM((_ROWS_PER_W * _K,), jnp.float32),
        pltpu.VMEM((_ROWS_PER_W * _K,), jnp.int32),
        pltpu.SemaphoreType.DMA,
        pltpu.SemaphoreType.DMA,
    ],
)(_sc_body)


def _tc_sort_body(k_ref, i_ref, vo_ref, io_ref):
    k = k_ref[...]
    ix = i_ref[...]
    lane = lax.broadcasted_iota(jnp.int32, (_NROWS, _K), 1)
    for kk in [2, 4, 8, 16, 32, 64]:
        j = kk // 2
        while j >= 1:
            lk = jnp.concatenate([k[:, j:], k[:, :j]], axis=1)
            rk = jnp.concatenate([k[:, _K - j:], k[:, :_K - j]], axis=1)
            li = jnp.concatenate([ix[:, j:], ix[:, :j]], axis=1)
            ri = jnp.concatenate([ix[:, _K - j:], ix[:, :_K - j]], axis=1)
            bitclear = (lane & j) == 0
            pk = jnp.where(bitclear, lk, rk)
            pi = jnp.where(bitclear, li, ri)
            first = (k > pk) | ((k == pk) & (ix < pi))
            forward = (lane & kk) == 0
            take = bitclear ^ first ^ (~forward)
            k = jnp.where(take, pk, k)
            ix = jnp.where(take, pi, ix)
            j //= 2
    vo_ref[...] = k
    io_ref[...] = ix


_tc_sort = pl.pallas_call(
    _tc_sort_body,
    out_shape=(
        jax.ShapeDtypeStruct((_NROWS, _K), jnp.float32),
        jax.ShapeDtypeStruct((_NROWS, _K), jnp.int32),
    ),
)


def kernel(x):
    wv, wi = _sc_topk(x)
    return _tc_sort(wv.reshape(_NROWS, _K), wi.reshape(_NROWS, _K))


# speedup vs baseline: 2.3601x; 1.0659x over previous
import functools

import jax
import jax.numpy as jnp
from jax import lax
from jax.experimental import pallas as pl
from jax.experimental.pallas import tpu as pltpu
from jax.experimental.pallas import tpu_sc as plsc

_K = 64
_NROWS = 128
_N = 32768
_LANES = 16
_NWORKERS = 32
_ROWS_PER_W = _NROWS // _NWORKERS
_CHUNKS = _N // _LANES


def _f32_key(v):
    b = plsc.bitcast(v, jnp.int32)
    return b ^ (jnp.right_shift(b, 31) & jnp.int32(0x7FFFFFFF))


def _sc_body(x_hbm, outv_hbm, outi_hbm, row_a, row_b, ci_v, h1_v, h2_v,
             suf_v, wk_v, wi_v, sem_a, sem_b):
    wid = lax.axis_index("s") * 2 + lax.axis_index("c")
    lanes = lax.iota(jnp.int32, _LANES)
    ones = jnp.ones((_LANES,), jnp.int32)
    zvec = jnp.zeros((_LANES,), jnp.int32)
    l1off = lanes * 256
    l2off = lanes * 17
    base_row = wid * _ROWS_PER_W

    bufs = (row_a, row_b)
    sems = (sem_a, sem_b)
    pending = pltpu.async_copy(x_hbm.at[base_row], row_a, sem_a)
    for j in range(_ROWS_PER_W):
        cur = bufs[j % 2]
        if j + 1 < _ROWS_PER_W:
            nxt = pltpu.async_copy(
                x_hbm.at[base_row + j + 1], bufs[(j + 1) % 2],
                sems[(j + 1) % 2])
        pending.wait()

        def cl1_body(i, _):
            h1_v[pl.ds(i * _LANES, _LANES)] = zvec
            return jnp.int32(0)

        lax.fori_loop(0, 256, cl1_body, jnp.int32(0))

        def h1_body(c0, _):
            for u in range(4):
                key = _f32_key(cur[pl.ds(c0 * 64 + u * _LANES, _LANES)])
                dig = jnp.right_shift(key, 24) + 128
                plsc.addupdate_scatter(h1_v, [l1off + dig], ones)
            return jnp.int32(0)

        lax.fori_loop(0, _CHUNKS // 4, h1_body, jnp.int32(0))

        def tot_body(g, _):
            def add_body(l, acc):
                return acc + h1_v[pl.ds(l * 256 + g * _LANES, _LANES)]

            acc = lax.fori_loop(0, _LANES, add_body, zvec)
            suf_v[pl.ds(g * _LANES, _LANES)] = acc
            return jnp.int32(0)

        lax.fori_loop(0, 16, tot_body, jnp.int32(0))

        def suf_body(g2, carry):
            g = 15 - g2
            v = suf_v[pl.ds(g * _LANES, _LANES)]
            s = lax.rev(jnp.cumsum(lax.rev(v, (0,))), (0,)) + carry
            suf_v[pl.ds(g * _LANES, _LANES)] = s
            return (carry + jnp.sum(v)).astype(jnp.int32)

        lax.fori_loop(0, 16, suf_body, jnp.int32(0))

        def find_body(g, b):
            bids = g * _LANES + lanes
            sv = suf_v[pl.ds(g * _LANES, _LANES)]
            cand = jnp.where(sv >= _K, bids, -1)
            return jnp.maximum(b, jnp.max(cand)).astype(jnp.int32)

        b1 = lax.fori_loop(0, 16, find_body, jnp.int32(-1))
        thresh = jnp.left_shift(b1 - 128, 24).astype(jnp.int32)

        def p2_body(c0, cn):
            for u in range(2):
                off = c0 * 32 + u * _LANES
                key = _f32_key(cur[pl.ds(off, _LANES)])
                keep = key >= thresh
                ki = keep.astype(jnp.int32)
                pos = cn + jnp.cumsum(ki) - ki
                plsc.store_scatter(ci_v, [pos], off + lanes, mask=keep)
                cn = cn + plsc.all_reduce_population_count(keep)
            return cn

        cn = lax.fori_loop(0, _CHUNKS // 2, p2_body, zvec)
        m = jnp.max(cn).astype(jnp.int32)
        c_gt = jnp.int32(0)

        def level_body(lvl, carry):
            thresh, m, _ = carry
            s = (20 - 4 * lvl).astype(jnp.int32)

            def cl2_body(i, _):
                h2_v[pl.ds(i * _LANES, _LANES)] = zvec
                return jnp.int32(0)

            lax.fori_loop(0, 17, cl2_body, jnp.int32(0))
            nch = (m + _LANES - 1) // _LANES
            width = jnp.left_shift(jnp.int32(1), s + 4)

            def hb_body(c, _):
                idx = ci_v[pl.ds(c * _LANES, _LANES)]
                valid = (c * _LANES + lanes) < m
                kv = plsc.load_gather(cur, [idx], mask=valid)
                key = _f32_key(kv)
                d = plsc.bitcast(key - thresh, jnp.uint32)
                bnd = valid & (d < plsc.bitcast(
                    jnp.full((_LANES,), width, jnp.int32), jnp.uint32))
                dig = jnp.where(
                    bnd, jnp.right_shift(d, s.astype(jnp.uint32))
                    .astype(jnp.int32), 16)
                plsc.addupdate_scatter(h2_v, [l2off + dig], ones)
                return jnp.int32(0)

            lax.fori_loop(0, nch, hb_body, jnp.int32(0))

            tot = zvec
            for l in range(_LANES):
                tot = tot + h2_v[pl.ds(l * 17, _LANES)]
            suffix = lax.rev(jnp.cumsum(lax.rev(tot, (0,))), (0,))
            nB = jnp.sum(tot).astype(jnp.int32)
            r_l = jnp.int32(_K) - (m - nB)
            b = jnp.max(jnp.where(suffix >= r_l, lanes, -1)).astype(jnp.int32)
            sufb1 = jnp.sum(jnp.where(lanes == b + 1, suffix, 0)).astype(
                jnp.int32)
            c_gt = (m - nB) + sufb1
            thresh = (thresh + jnp.left_shift(b, s)).astype(jnp.int32)

            def cb_body(c, cn2):
                idx = ci_v[pl.ds(c * _LANES, _LANES)]
                valid = (c * _LANES + lanes) < m
                kv = plsc.load_gather(cur, [idx], mask=valid)
                key = _f32_key(kv)
                keep = valid & (key >= thresh)
                ki = keep.astype(jnp.int32)
                pos = cn2 + jnp.cumsum(ki) - ki
                plsc.store_scatter(ci_v, [pos], idx, mask=keep)
                return cn2 + plsc.all_reduce_population_count(keep)

            cn2 = lax.fori_loop(0, nch, cb_body, zvec)
            return (thresh, jnp.max(cn2).astype(jnp.int32), c_gt)

        thresh, m, c_gt = lax.fori_loop(
            0, 6, level_body, (thresh, m, c_gt))

        r_fin = jnp.int32(_K) - c_gt
        nchf = (m + _LANES - 1) // _LANES

        def f_body(c, carry):
            neq, nw = carry
            idx = ci_v[pl.ds(c * _LANES, _LANES)]
            valid = (c * _LANES + lanes) < m
            kv = plsc.load_gather(cur, [idx], mask=valid)
            key = _f32_key(kv)
            gt = valid & (key > thresh)
            eq = valid & (key == thresh)
            eqi = eq.astype(jnp.int32)
            tier = neq + jnp.cumsum(eqi) - eqi
            keep = gt | (eq & (tier < r_fin))
            ki = keep.astype(jnp.int32)
            pos = nw + jnp.cumsum(ki) - ki + (j * _K)
            plsc.store_scatter(wi_v, [pos], idx, mask=keep)
            return (neq + plsc.all_reduce_population_count(eq),
                    nw + plsc.all_reduce_population_count(keep))

        lax.fori_loop(0, nchf, f_body, (zvec, zvec))

        for c in range(_K // _LANES):
            iv = wi_v[pl.ds(j * _K + c * _LANES, _LANES)]
            wk_v[pl.ds(j * _K + c * _LANES, _LANES)] = plsc.load_gather(
                cur, [iv])

        if j + 1 < _ROWS_PER_W:
            pending = nxt

    pltpu.sync_copy(wk_v, outv_hbm.at[pl.ds(wid * (_ROWS_PER_W * _K),
                                            _ROWS_PER_W * _K)])
    pltpu.sync_copy(wi_v, outi_hbm.at[pl.ds(wid * (_ROWS_PER_W * _K),
                                            _ROWS_PER_W * _K)])


_sc_topk = functools.partial(
    pl.kernel,
    out_type=(
        jax.ShapeDtypeStruct((_NROWS * _K,), jnp.float32),
        jax.ShapeDtypeStruct((_NROWS * _K,), jnp.int32),
    ),
    mesh=plsc.VectorSubcoreMesh(core_axis_name="c", subcore_axis_name="s"),
    compiler_params=pltpu.CompilerParams(
        needs_layout_passes=False, use_tc_tiling_on_sc=True),
    scratch_types=[
        pltpu.VMEM((_N,), jnp.float32),
        pltpu.VMEM((_N,), jnp.float32),
        pltpu.VMEM((_N,), jnp.int32),
        pltpu.VMEM((16 * 256,), jnp.int32),
        pltpu.VMEM((16 * 17,), jnp.int32),
        pltpu.VMEM((256,), jnp.int32),
        pltpu.VMEM((_ROWS_PER_W * _K,), jnp.float32),
        pltpu.VMEM((_ROWS_PER_W * _K,), jnp.int32),
        pltpu.SemaphoreType.DMA,
        pltpu.SemaphoreType.DMA,
    ],
)(_sc_body)


def _tc_sort_body(k_ref, i_ref, vo_ref, io_ref):
    k = k_ref[...]
    ix = i_ref[...]
    lane = lax.broadcasted_iota(jnp.int32, (_NROWS, _K), 1)
    for kk in [2, 4, 8, 16, 32, 64]:
        j = kk // 2
        while j >= 1:
            lk = jnp.concatenate([k[:, j:], k[:, :j]], axis=1)
            rk = jnp.concatenate([k[:, _K - j:], k[:, :_K - j]], axis=1)
            li = jnp.concatenate([ix[:, j:], ix[:, :j]], axis=1)
            ri = jnp.concatenate([ix[:, _K - j:], ix[:, :_K - j]], axis=1)
            bitclear = (lane & j) == 0
            pk = jnp.where(bitclear, lk, rk)
            pi = jnp.where(bitclear, li, ri)
            first = (k > pk) | ((k == pk) & (ix < pi))
            forward = (lane & kk) == 0
            take = bitclear ^ first ^ (~forward)
            k = jnp.where(take, pk, k)
            ix = jnp.where(take, pi, ix)
            j //= 2
    vo_ref[...] = k
    io_ref[...] = ix


_tc_sort = pl.pallas_call(
    _tc_sort_body,
    out_shape=(
        jax.ShapeDtypeStruct((_NROWS, _K), jnp.float32),
        jax.ShapeDtypeStruct((_NROWS, _K), jnp.int32),
    ),
)


def kernel(x):
    wv, wi = _sc_topk(x)
    return _tc_sort(wv.reshape(_NROWS, _K), wi.reshape(_NROWS, _K))


# speedup vs baseline: 5.3199x; 2.2541x over previous
import functools

import jax
import jax.numpy as jnp
from jax import lax
from jax.experimental import pallas as pl
from jax.experimental.pallas import tpu as pltpu
from jax.experimental.pallas import tpu_sc as plsc

_K = 64
_NROWS = 128
_N = 32768
_LANES = 16
_NWORKERS = 32
_ROWS_PER_W = _NROWS // _NWORKERS
_CHUNKS = _N // _LANES


def _f32_key(v):
    b = plsc.bitcast(v, jnp.int32)
    return b ^ (jnp.right_shift(b, 31) & jnp.int32(0x7FFFFFFF))


def _sc_body(x_hbm, outv_hbm, outi_hbm, row_a, row_b, ci_v, h1_v, h2_v,
             suf_v, wk_v, wi_v, sem_a, sem_b):
    wid = lax.axis_index("s") * 2 + lax.axis_index("c")
    lanes = lax.iota(jnp.int32, _LANES)
    ones = jnp.ones((_LANES,), jnp.int32)
    zvec = jnp.zeros((_LANES,), jnp.int32)
    l1off = lanes * 256
    l2off = lanes * 17
    base_row = wid * _ROWS_PER_W

    bufs = (row_a, row_b)
    sems = (sem_a, sem_b)
    pending = pltpu.async_copy(x_hbm.at[base_row], row_a, sem_a)
    for j in range(_ROWS_PER_W):
        cur = bufs[j % 2]
        if j + 1 < _ROWS_PER_W:
            nxt = pltpu.async_copy(
                x_hbm.at[base_row + j + 1], bufs[(j + 1) % 2],
                sems[(j + 1) % 2])
        pending.wait()

        def cl1_body(i, _):
            h1_v[pl.ds(i * _LANES, _LANES)] = zvec
            return jnp.int32(0)

        lax.fori_loop(0, 256, cl1_body, jnp.int32(0))

        def h1_body(c0, _):
            base = c0 * (_LANES * 8)
            vs = [cur[pl.ds(base + u * _LANES, _LANES)] for u in range(8)]
            bs = [plsc.bitcast(v, jnp.int32) for v in vs]
            sg = [jnp.right_shift(b, 31) for b in bs]
            fl = [s | jnp.int32(-2147483648) for s in sg]
            us = [plsc.bitcast(b ^ f, jnp.uint32) for b, f in zip(bs, fl)]
            dg = [plsc.bitcast(jnp.right_shift(u, 24), jnp.int32) for u in us]
            ad = [l1off + d for d in dg]
            for a in ad:
                plsc.addupdate_scatter(h1_v, [a], ones)
            return jnp.int32(0)

        lax.fori_loop(0, _CHUNKS // 8, h1_body, jnp.int32(0))

        def tot_body(l, accs):
            return tuple(
                accs[g] + h1_v[pl.ds(l * 256 + g * _LANES, _LANES)]
                for g in range(16))

        accs = lax.fori_loop(0, _LANES, tot_body, (zvec,) * 16)
        for g in range(16):
            suf_v[pl.ds(g * _LANES, _LANES)] = accs[g]

        def suf_body(g2, carry):
            g = 15 - g2
            v = suf_v[pl.ds(g * _LANES, _LANES)]
            s = lax.rev(jnp.cumsum(lax.rev(v, (0,))), (0,)) + carry
            suf_v[pl.ds(g * _LANES, _LANES)] = s
            return (carry + jnp.sum(v)).astype(jnp.int32)

        lax.fori_loop(0, 16, suf_body, jnp.int32(0))

        def find_body(g, b):
            bids = g * _LANES + lanes
            sv = suf_v[pl.ds(g * _LANES, _LANES)]
            cand = jnp.where(sv >= _K, bids, -1)
            return jnp.maximum(b, jnp.max(cand)).astype(jnp.int32)

        b1 = lax.fori_loop(0, 16, find_body, jnp.int32(-1))
        thresh = jnp.left_shift(b1 - 128, 24).astype(jnp.int32)

        def p2_body(c0, cn):
            base = c0 * (_LANES * 8)
            offs = [base + u * _LANES for u in range(8)]
            vs = [cur[pl.ds(o, _LANES)] for o in offs]
            bs = [plsc.bitcast(v, jnp.int32) for v in vs]
            sg = [jnp.right_shift(b, 31) for b in bs]
            ks = [b ^ (s & jnp.int32(0x7FFFFFFF)) for b, s in zip(bs, sg)]
            kp = [k >= thresh for k in ks]
            ki = [k.astype(jnp.int32) for k in kp]
            cs = [jnp.cumsum(x) for x in ki]
            pc = [plsc.all_reduce_population_count(k) for k in kp]
            cns = [cn]
            for u in range(8):
                cns.append(cns[-1] + pc[u])
            pos = [cns[u] + cs[u] - ki[u] for u in range(8)]
            for u in range(8):
                plsc.store_scatter(ci_v, [pos[u]], offs[u] + lanes,
                                   mask=kp[u])
            return cns[8]

        cn = lax.fori_loop(0, _CHUNKS // 8, p2_body, zvec)
        m = jnp.max(cn).astype(jnp.int32)
        c_gt = jnp.int32(0)

        def level_body(lvl, carry):
            thresh, m, _ = carry
            s = (20 - 4 * lvl).astype(jnp.int32)

            def cl2_body(i, _):
                h2_v[pl.ds(i * _LANES, _LANES)] = zvec
                return jnp.int32(0)

            lax.fori_loop(0, 17, cl2_body, jnp.int32(0))
            nch4 = (m + _LANES * 4 - 1) // (_LANES * 4)
            width_u = plsc.bitcast(
                jnp.full((_LANES,), jnp.left_shift(jnp.int32(1), s + 4),
                         jnp.int32), jnp.uint32)
            s_u = plsc.bitcast(jnp.full((_LANES,), s, jnp.int32), jnp.uint32)

            def hb_body(c0, _):
                base = c0 * (_LANES * 4)
                offs = [base + u * _LANES for u in range(4)]
                idxs = [ci_v[pl.ds(o, _LANES)] for o in offs]
                vls = [(o + lanes) < m for o in offs]
                kvs = [plsc.load_gather(cur, [i], mask=v)
                       for i, v in zip(idxs, vls)]
                kys = [_f32_key(kv) for kv in kvs]
                ds_ = [plsc.bitcast(k - thresh, jnp.uint32) for k in kys]
                bnd = [v & (d < width_u) for v, d in zip(vls, ds_)]
                dgs = [jnp.where(b, plsc.bitcast(jnp.right_shift(d, s_u),
                                                 jnp.int32), 16)
                       for b, d in zip(bnd, ds_)]
                for d in dgs:
                    plsc.addupdate_scatter(h2_v, [l2off + d], ones)
                return jnp.int32(0)

            lax.fori_loop(0, nch4, hb_body, jnp.int32(0))

            tot = zvec
            for l in range(_LANES):
                tot = tot + h2_v[pl.ds(l * 17, _LANES)]
            suffix = lax.rev(jnp.cumsum(lax.rev(tot, (0,))), (0,))
            nB = jnp.sum(tot).astype(jnp.int32)
            r_l = jnp.int32(_K) - (m - nB)
            b = jnp.max(jnp.where(suffix >= r_l, lanes, -1)).astype(jnp.int32)
            sufb1 = jnp.sum(jnp.where(lanes == b + 1, suffix, 0)).astype(
                jnp.int32)
            c_gt = (m - nB) + sufb1
            thresh = (thresh + jnp.left_shift(b, s)).astype(jnp.int32)

            def cb_body(c0, cn2):
                base = c0 * (_LANES * 4)
                offs = [base + u * _LANES for u in range(4)]
                idxs = [ci_v[pl.ds(o, _LANES)] for o in offs]
                vls = [(o + lanes) < m for o in offs]
                kvs = [plsc.load_gather(cur, [i], mask=v)
                       for i, v in zip(idxs, vls)]
                kys = [_f32_key(kv) for kv in kvs]
                kp = [v & (k >= thresh) for v, k in zip(vls, kys)]
                ki = [k.astype(jnp.int32) for k in kp]
                cs = [jnp.cumsum(x) for x in ki]
                pc = [plsc.all_reduce_population_count(k) for k in kp]
                cns = [cn2]
                for u in range(4):
                    cns.append(cns[-1] + pc[u])
                pos = [cns[u] + cs[u] - ki[u] for u in range(4)]
                for u in range(4):
                    plsc.store_scatter(ci_v, [pos[u]], idxs[u], mask=kp[u])
                return cns[4]

            cn2 = lax.fori_loop(0, nch4, cb_body, zvec)
            return (thresh, jnp.max(cn2).astype(jnp.int32), c_gt)

        thresh, m, c_gt = lax.fori_loop(
            0, 6, level_body, (thresh, m, c_gt))

        r_fin = jnp.int32(_K) - c_gt
        nchf = (m + _LANES - 1) // _LANES

        def f_body(c, carry):
            neq, nw = carry
            idx = ci_v[pl.ds(c * _LANES, _LANES)]
            valid = (c * _LANES + lanes) < m
            kv = plsc.load_gather(cur, [idx], mask=valid)
            key = _f32_key(kv)
            gt = valid & (key > thresh)
            eq = valid & (key == thresh)
            eqi = eq.astype(jnp.int32)
            tier = neq + jnp.cumsum(eqi) - eqi
            keep = gt | (eq & (tier < r_fin))
            ki = keep.astype(jnp.int32)
            pos = nw + jnp.cumsum(ki) - ki + (j * _K)
            plsc.store_scatter(wi_v, [pos], idx, mask=keep)
            return (neq + plsc.all_reduce_population_count(eq),
                    nw + plsc.all_reduce_population_count(keep))

        lax.fori_loop(0, nchf, f_body, (zvec, zvec))

        for c in range(_K // _LANES):
            iv = wi_v[pl.ds(j * _K + c * _LANES, _LANES)]
            wk_v[pl.ds(j * _K + c * _LANES, _LANES)] = plsc.load_gather(
                cur, [iv])

        if j + 1 < _ROWS_PER_W:
            pending = nxt

    pltpu.sync_copy(wk_v, outv_hbm.at[pl.ds(wid * (_ROWS_PER_W * _K),
                                            _ROWS_PER_W * _K)])
    pltpu.sync_copy(wi_v, outi_hbm.at[pl.ds(wid * (_ROWS_PER_W * _K),
                                            _ROWS_PER_W * _K)])


_sc_topk = functools.partial(
    pl.kernel,
    out_type=(
        jax.ShapeDtypeStruct((_NROWS * _K,), jnp.float32),
        jax.ShapeDtypeStruct((_NROWS * _K,), jnp.int32),
    ),
    mesh=plsc.VectorSubcoreMesh(core_axis_name="c", subcore_axis_name="s"),
    compiler_params=pltpu.CompilerParams(
        needs_layout_passes=False, use_tc_tiling_on_sc=True),
    scratch_types=[
        pltpu.VMEM((_N,), jnp.float32),
        pltpu.VMEM((_N,), jnp.float32),
        pltpu.VMEM((_N,), jnp.int32),
        pltpu.VMEM((16 * 256,), jnp.int32),
        pltpu.VMEM((16 * 17,), jnp.int32),
        pltpu.VMEM((256,), jnp.int32),
        pltpu.VMEM((_ROWS_PER_W * _K,), jnp.float32),
        pltpu.VMEM((_ROWS_PER_W * _K,), jnp.int32),
        pltpu.SemaphoreType.DMA,
        pltpu.SemaphoreType.DMA,
    ],
)(_sc_body)


def _tc_sort_body(k_ref, i_ref, vo_ref, io_ref):
    k = k_ref[...]
    ix = i_ref[...]
    lane = lax.broadcasted_iota(jnp.int32, (_NROWS, _K), 1)
    for kk in [2, 4, 8, 16, 32, 64]:
        j = kk // 2
        while j >= 1:
            lk = jnp.concatenate([k[:, j:], k[:, :j]], axis=1)
            rk = jnp.concatenate([k[:, _K - j:], k[:, :_K - j]], axis=1)
            li = jnp.concatenate([ix[:, j:], ix[:, :j]], axis=1)
            ri = jnp.concatenate([ix[:, _K - j:], ix[:, :_K - j]], axis=1)
            bitclear = (lane & j) == 0
            pk = jnp.where(bitclear, lk, rk)
            pi = jnp.where(bitclear, li, ri)
            first = (k > pk) | ((k == pk) & (ix < pi))
            forward = (lane & kk) == 0
            take = bitclear ^ first ^ (~forward)
            k = jnp.where(take, pk, k)
            ix = jnp.where(take, pi, ix)
            j //= 2
    vo_ref[...] = k
    io_ref[...] = ix


_tc_sort = pl.pallas_call(
    _tc_sort_body,
    out_shape=(
        jax.ShapeDtypeStruct((_NROWS, _K), jnp.float32),
        jax.ShapeDtypeStruct((_NROWS, _K), jnp.int32),
    ),
)


def kernel(x):
    wv, wi = _sc_topk(x)
    return _tc_sort(wv.reshape(_NROWS, _K), wi.reshape(_NROWS, _K))


# speedup vs baseline: 5.9406x; 1.1167x over previous
import functools

import jax
import jax.numpy as jnp
from jax import lax
from jax.experimental import pallas as pl
from jax.experimental.pallas import tpu as pltpu
from jax.experimental.pallas import tpu_sc as plsc

_K = 64
_NROWS = 128
_N = 32768
_LANES = 16
_NWORKERS = 32
_ROWS_PER_W = _NROWS // _NWORKERS
_CHUNKS = _N // _LANES


def _f32_key(v):
    b = plsc.bitcast(v, jnp.int32)
    return b ^ (jnp.right_shift(b, 31) & jnp.int32(0x7FFFFFFF))


def _sc_body(x_hbm, outv_hbm, outi_hbm, row_a, row_b, ci_v, h1_v, h2_v,
             suf_v, wk_v, wi_v, sem_a, sem_b):
    wid = lax.axis_index("s") * 2 + lax.axis_index("c")
    lanes = lax.iota(jnp.int32, _LANES)
    ones = jnp.ones((_LANES,), jnp.int32)
    zvec = jnp.zeros((_LANES,), jnp.int32)
    l1off = lanes * 256
    l2off = lanes * 17
    base_row = wid * _ROWS_PER_W

    bufs = (row_a, row_b)
    sems = (sem_a, sem_b)
    pending = pltpu.async_copy(x_hbm.at[base_row], row_a, sem_a)

    def cl1_body(i, _):
        h1_v[pl.ds(i * _LANES, _LANES)] = zvec
        return jnp.int32(0)

    lax.fori_loop(0, 256, cl1_body, jnp.int32(0))

    def cl2_body(i, _):
        h2_v[pl.ds(i * _LANES, _LANES)] = zvec
        return jnp.int32(0)

    lax.fori_loop(0, 17, cl2_body, jnp.int32(0))

    for j in range(_ROWS_PER_W):
        cur = bufs[j % 2]
        if j + 1 < _ROWS_PER_W:
            nxt = pltpu.async_copy(
                x_hbm.at[base_row + j + 1], bufs[(j + 1) % 2],
                sems[(j + 1) % 2])
        pending.wait()


        def h1_body(c0, _):
            base = c0 * (_LANES * 8)
            vs = [cur[pl.ds(base + u * _LANES, _LANES)] for u in range(8)]
            bs = [plsc.bitcast(v, jnp.int32) for v in vs]
            sg = [jnp.right_shift(b, 31) for b in bs]
            fl = [s | jnp.int32(-2147483648) for s in sg]
            us = [plsc.bitcast(b ^ f, jnp.uint32) for b, f in zip(bs, fl)]
            dg = [plsc.bitcast(jnp.right_shift(u, 24), jnp.int32) for u in us]
            ad = [l1off + d for d in dg]
            for a in ad:
                plsc.addupdate_scatter(h1_v, [a], ones)
            return jnp.int32(0)

        lax.fori_loop(0, _CHUNKS // 8, h1_body, jnp.int32(0))

        def tot_body(l, accs):
            loaded = [h1_v[pl.ds(l * 256 + g * _LANES, _LANES)]
                      for g in range(16)]
            for g in range(16):
                h1_v[pl.ds(l * 256 + g * _LANES, _LANES)] = zvec
            return tuple(accs[g] + loaded[g] for g in range(16))

        accs = lax.fori_loop(0, _LANES, tot_body, (zvec,) * 16)
        for g in range(16):
            suf_v[pl.ds(g * _LANES, _LANES)] = accs[g]

        def suf_body(g2, carry):
            g = 15 - g2
            v = suf_v[pl.ds(g * _LANES, _LANES)]
            s = lax.rev(jnp.cumsum(lax.rev(v, (0,))), (0,)) + carry
            suf_v[pl.ds(g * _LANES, _LANES)] = s
            return (carry + jnp.sum(v)).astype(jnp.int32)

        lax.fori_loop(0, 16, suf_body, jnp.int32(0))

        def find_body(g, b):
            bids = g * _LANES + lanes
            sv = suf_v[pl.ds(g * _LANES, _LANES)]
            cand = jnp.where(sv >= _K, bids, -1)
            return jnp.maximum(b, jnp.max(cand)).astype(jnp.int32)

        b1 = lax.fori_loop(0, 16, find_body, jnp.int32(-1))
        thresh = jnp.left_shift(b1 - 128, 24).astype(jnp.int32)

        def p2_body(c0, cnm1):
            base = c0 * (_LANES * 8)
            offs = [base + u * _LANES for u in range(8)]
            vs = [cur[pl.ds(o, _LANES)] for o in offs]
            bs = [plsc.bitcast(v, jnp.int32) for v in vs]
            sg = [jnp.right_shift(b, 31) for b in bs]
            ks = [b ^ (s & jnp.int32(0x7FFFFFFF)) for b, s in zip(bs, sg)]
            kp = [k >= thresh for k in ks]
            cs = [plsc.cumsum(ones, mask=k) for k in kp]
            pc = [plsc.all_reduce_population_count(k) for k in kp]
            cns = [cnm1]
            for u in range(8):
                cns.append(cns[-1] + pc[u])
            pos = [cns[u] + cs[u] for u in range(8)]
            for u in range(8):
                plsc.store_scatter(ci_v, [pos[u]], offs[u] + lanes,
                                   mask=kp[u])
            return cns[8]

        cn = lax.fori_loop(0, _CHUNKS // 8, p2_body, zvec - 1)
        m = (jnp.max(cn) + 1).astype(jnp.int32)
        c_gt = jnp.int32(0)

        def level_body(lvl, carry):
            thresh, m, _ = carry
            s = (20 - 4 * lvl).astype(jnp.int32)
            nch4 = (m + _LANES * 4 - 1) // (_LANES * 4)
            width_u = plsc.bitcast(
                jnp.full((_LANES,), jnp.left_shift(jnp.int32(1), s + 4),
                         jnp.int32), jnp.uint32)
            s_u = plsc.bitcast(jnp.full((_LANES,), s, jnp.int32), jnp.uint32)

            def hb_body(c0, _):
                base = c0 * (_LANES * 4)
                offs = [base + u * _LANES for u in range(4)]
                idxs = [ci_v[pl.ds(o, _LANES)] for o in offs]
                vls = [(o + lanes) < m for o in offs]
                kvs = [plsc.load_gather(cur, [i], mask=v)
                       for i, v in zip(idxs, vls)]
                kys = [_f32_key(kv) for kv in kvs]
                ds_ = [plsc.bitcast(k - thresh, jnp.uint32) for k in kys]
                bnd = [v & (d < width_u) for v, d in zip(vls, ds_)]
                dgs = [jnp.where(b, plsc.bitcast(jnp.right_shift(d, s_u),
                                                 jnp.int32), 16)
                       for b, d in zip(bnd, ds_)]
                for d in dgs:
                    plsc.addupdate_scatter(h2_v, [l2off + d], ones)
                return jnp.int32(0)

            lax.fori_loop(0, nch4, hb_body, jnp.int32(0))

            loaded = [h2_v[pl.ds(l * 17, _LANES)] for l in range(_LANES)]
            for l in range(_LANES):
                h2_v[pl.ds(l * 17, _LANES)] = zvec
            tot = loaded[0]
            for l in range(1, _LANES):
                tot = tot + loaded[l]
            suffix = lax.rev(jnp.cumsum(lax.rev(tot, (0,))), (0,))
            nB = jnp.sum(tot).astype(jnp.int32)
            r_l = jnp.int32(_K) - (m - nB)
            b = jnp.max(jnp.where(suffix >= r_l, lanes, -1)).astype(jnp.int32)
            sufb1 = jnp.sum(jnp.where(lanes == b + 1, suffix, 0)).astype(
                jnp.int32)
            c_gt = (m - nB) + sufb1
            thresh = (thresh + jnp.left_shift(b, s)).astype(jnp.int32)

            def cb_body(c0, cnm1):
                base = c0 * (_LANES * 4)
                offs = [base + u * _LANES for u in range(4)]
                idxs = [ci_v[pl.ds(o, _LANES)] for o in offs]
                vls = [(o + lanes) < m for o in offs]
                kvs = [plsc.load_gather(cur, [i], mask=v)
                       for i, v in zip(idxs, vls)]
                kys = [_f32_key(kv) for kv in kvs]
                kp = [v & (k >= thresh) for v, k in zip(vls, kys)]
                cs = [plsc.cumsum(ones, mask=k) for k in kp]
                pc = [plsc.all_reduce_population_count(k) for k in kp]
                cns = [cnm1]
                for u in range(4):
                    cns.append(cns[-1] + pc[u])
                pos = [cns[u] + cs[u] for u in range(4)]
                for u in range(4):
                    plsc.store_scatter(ci_v, [pos[u]], idxs[u], mask=kp[u])
                return cns[4]

            cn2 = lax.fori_loop(0, nch4, cb_body, zvec - 1)
            return (thresh, (jnp.max(cn2) + 1).astype(jnp.int32), c_gt)

        thresh, m, c_gt = lax.fori_loop(
            0, 6, level_body, (thresh, m, c_gt))

        r_fin = jnp.int32(_K) - c_gt
        nchf = (m + _LANES - 1) // _LANES

        def f_body(c, carry):
            neq, nw = carry
            idx = ci_v[pl.ds(c * _LANES, _LANES)]
            valid = (c * _LANES + lanes) < m
            kv = plsc.load_gather(cur, [idx], mask=valid)
            key = _f32_key(kv)
            gt = valid & (key > thresh)
            eq = valid & (key == thresh)
            eqi = eq.astype(jnp.int32)
            tier = neq + jnp.cumsum(eqi) - eqi
            keep = gt | (eq & (tier < r_fin))
            ki = keep.astype(jnp.int32)
            pos = nw + jnp.cumsum(ki) - ki + (j * _K)
            plsc.store_scatter(wi_v, [pos], idx, mask=keep)
            return (neq + plsc.all_reduce_population_count(eq),
                    nw + plsc.all_reduce_population_count(keep))

        lax.fori_loop(0, nchf, f_body, (zvec, zvec))

        for c in range(_K // _LANES):
            iv = wi_v[pl.ds(j * _K + c * _LANES, _LANES)]
            wk_v[pl.ds(j * _K + c * _LANES, _LANES)] = plsc.load_gather(
                cur, [iv])

        if j + 1 < _ROWS_PER_W:
            pending = nxt

    pltpu.sync_copy(wk_v, outv_hbm.at[pl.ds(wid * (_ROWS_PER_W * _K),
                                            _ROWS_PER_W * _K)])
    pltpu.sync_copy(wi_v, outi_hbm.at[pl.ds(wid * (_ROWS_PER_W * _K),
                                            _ROWS_PER_W * _K)])


_sc_topk = functools.partial(
    pl.kernel,
    out_type=(
        jax.ShapeDtypeStruct((_NROWS * _K,), jnp.float32),
        jax.ShapeDtypeStruct((_NROWS * _K,), jnp.int32),
    ),
    mesh=plsc.VectorSubcoreMesh(core_axis_name="c", subcore_axis_name="s"),
    compiler_params=pltpu.CompilerParams(
        needs_layout_passes=False, use_tc_tiling_on_sc=True),
    scratch_types=[
        pltpu.VMEM((_N,), jnp.float32),
        pltpu.VMEM((_N,), jnp.float32),
        pltpu.VMEM((_N,), jnp.int32),
        pltpu.VMEM((16 * 256,), jnp.int32),
        pltpu.VMEM((16 * 17,), jnp.int32),
        pltpu.VMEM((256,), jnp.int32),
        pltpu.VMEM((_ROWS_PER_W * _K,), jnp.float32),
        pltpu.VMEM((_ROWS_PER_W * _K,), jnp.int32),
        pltpu.SemaphoreType.DMA,
        pltpu.SemaphoreType.DMA,
    ],
)(_sc_body)


def _tc_sort_body(k_ref, i_ref, vo_ref, io_ref):
    k = k_ref[...]
    ix = i_ref[...]
    lane = lax.broadcasted_iota(jnp.int32, (_NROWS, _K), 1)
    for kk in [2, 4, 8, 16, 32, 64]:
        j = kk // 2
        while j >= 1:
            lk = jnp.concatenate([k[:, j:], k[:, :j]], axis=1)
            rk = jnp.concatenate([k[:, _K - j:], k[:, :_K - j]], axis=1)
            li = jnp.concatenate([ix[:, j:], ix[:, :j]], axis=1)
            ri = jnp.concatenate([ix[:, _K - j:], ix[:, :_K - j]], axis=1)
            bitclear = (lane & j) == 0
            pk = jnp.where(bitclear, lk, rk)
            pi = jnp.where(bitclear, li, ri)
            first = (k > pk) | ((k == pk) & (ix < pi))
            forward = (lane & kk) == 0
            take = bitclear ^ first ^ (~forward)
            k = jnp.where(take, pk, k)
            ix = jnp.where(take, pi, ix)
            j //= 2
    vo_ref[...] = k
    io_ref[...] = ix


_tc_sort = pl.pallas_call(
    _tc_sort_body,
    out_shape=(
        jax.ShapeDtypeStruct((_NROWS, _K), jnp.float32),
        jax.ShapeDtypeStruct((_NROWS, _K), jnp.int32),
    ),
)


def kernel(x):
    wv, wi = _sc_topk(x)
    return _tc_sort(wv.reshape(_NROWS, _K), wi.reshape(_NROWS, _K))


# speedup vs baseline: 6.0555x; 1.0193x over previous
import functools

import jax
import jax.numpy as jnp
from jax import lax
from jax.experimental import pallas as pl
from jax.experimental.pallas import tpu as pltpu
from jax.experimental.pallas import tpu_sc as plsc

_K = 64
_NROWS = 128
_N = 32768
_LANES = 16
_NWORKERS = 32
_ROWS_PER_W = _NROWS // _NWORKERS
_CHUNKS = _N // _LANES


def _f32_key(v):
    b = plsc.bitcast(v, jnp.int32)
    return b ^ (jnp.right_shift(b, 31) & jnp.int32(0x7FFFFFFF))


def _sc_body(x_hbm, outv_hbm, outi_hbm, row_a, row_b, ci_v, h1_v, h2_v,
             suf_v, wk_v, wi_v, sem_a, sem_b):
    wid = lax.axis_index("s") * 2 + lax.axis_index("c")
    lanes = lax.iota(jnp.int32, _LANES)
    ones = jnp.ones((_LANES,), jnp.int32)
    zvec = jnp.zeros((_LANES,), jnp.int32)
    l1off = lanes * 256
    l2off = lanes * 17
    base_row = wid * _ROWS_PER_W

    bufs = (row_a, row_b)
    sems = (sem_a, sem_b)
    pending = pltpu.async_copy(x_hbm.at[base_row], row_a, sem_a)

    def cl1_body(i, _):
        h1_v[pl.ds(i * _LANES, _LANES)] = zvec
        return jnp.int32(0)

    lax.fori_loop(0, 256, cl1_body, jnp.int32(0))

    def cl2_body(i, _):
        h2_v[pl.ds(i * _LANES, _LANES)] = zvec
        return jnp.int32(0)

    lax.fori_loop(0, 17, cl2_body, jnp.int32(0))

    for j in range(_ROWS_PER_W):
        cur = bufs[j % 2]
        if j + 1 < _ROWS_PER_W:
            nxt = pltpu.async_copy(
                x_hbm.at[base_row + j + 1], bufs[(j + 1) % 2],
                sems[(j + 1) % 2])
        pending.wait()


        def h1_body(c0, _):
            base = c0 * (_LANES * 16)
            vs = [cur[pl.ds(base + u * _LANES, _LANES)] for u in range(16)]
            bs = [plsc.bitcast(v, jnp.int32) for v in vs]
            sg = [jnp.right_shift(b, 31) for b in bs]
            fl = [s | jnp.int32(-2147483648) for s in sg]
            us = [plsc.bitcast(b ^ f, jnp.uint32) for b, f in zip(bs, fl)]
            dg = [plsc.bitcast(jnp.right_shift(u, 24), jnp.int32) for u in us]
            ad = [l1off + d for d in dg]
            for a in ad:
                plsc.addupdate_scatter(h1_v, [a], ones)
            return jnp.int32(0)

        lax.fori_loop(0, _CHUNKS // 16, h1_body, jnp.int32(0))

        def tot_body(l, accs):
            loaded = [h1_v[pl.ds(l * 256 + g * _LANES, _LANES)]
                      for g in range(16)]
            for g in range(16):
                h1_v[pl.ds(l * 256 + g * _LANES, _LANES)] = zvec
            return tuple(accs[g] + loaded[g] for g in range(16))

        accs = lax.fori_loop(0, _LANES, tot_body, (zvec,) * 16)
        for g in range(16):
            suf_v[pl.ds(g * _LANES, _LANES)] = accs[g]

        def suf_body(g2, carry):
            g = 15 - g2
            v = suf_v[pl.ds(g * _LANES, _LANES)]
            s = lax.rev(jnp.cumsum(lax.rev(v, (0,))), (0,)) + carry
            suf_v[pl.ds(g * _LANES, _LANES)] = s
            return (carry + jnp.sum(v)).astype(jnp.int32)

        lax.fori_loop(0, 16, suf_body, jnp.int32(0))

        def find_body(g, b):
            bids = g * _LANES + lanes
            sv = suf_v[pl.ds(g * _LANES, _LANES)]
            cand = jnp.where(sv >= _K, bids, -1)
            return jnp.maximum(b, jnp.max(cand)).astype(jnp.int32)

        b1 = lax.fori_loop(0, 16, find_body, jnp.int32(-1))
        thresh = jnp.left_shift(b1 - 128, 24).astype(jnp.int32)

        def p2_body(c0, cnm1):
            base = c0 * (_LANES * 8)
            offs = [base + u * _LANES for u in range(8)]
            vs = [cur[pl.ds(o, _LANES)] for o in offs]
            bs = [plsc.bitcast(v, jnp.int32) for v in vs]
            sg = [jnp.right_shift(b, 31) for b in bs]
            ks = [b ^ (s & jnp.int32(0x7FFFFFFF)) for b, s in zip(bs, sg)]
            kp = [k >= thresh for k in ks]
            cs = [plsc.cumsum(ones, mask=k) for k in kp]
            pc = [plsc.all_reduce_population_count(k) for k in kp]
            cns = [cnm1]
            for u in range(8):
                cns.append(cns[-1] + pc[u])
            pos = [cns[u] + cs[u] for u in range(8)]
            for u in range(8):
                plsc.store_scatter(ci_v, [pos[u]], offs[u] + lanes,
                                   mask=kp[u])
            return cns[8]

        cn = lax.fori_loop(0, _CHUNKS // 8, p2_body, zvec - 1)
        m = (jnp.max(cn) + 1).astype(jnp.int32)
        c_gt = jnp.int32(0)

        def level_body(lvl, carry):
            thresh, m, _ = carry
            s = (20 - 4 * lvl).astype(jnp.int32)
            nch8 = (m + _LANES * 8 - 1) // (_LANES * 8)
            width_u = plsc.bitcast(
                jnp.full((_LANES,), jnp.left_shift(jnp.int32(1), s + 4),
                         jnp.int32), jnp.uint32)
            s_u = plsc.bitcast(jnp.full((_LANES,), s, jnp.int32), jnp.uint32)

            def hb_body(c0, _):
                base = c0 * (_LANES * 8)
                offs = [base + u * _LANES for u in range(8)]
                idxs = [ci_v[pl.ds(o, _LANES)] for o in offs]
                vls = [(o + lanes) < m for o in offs]
                kvs = [plsc.load_gather(cur, [i], mask=v)
                       for i, v in zip(idxs, vls)]
                kys = [_f32_key(kv) for kv in kvs]
                ds_ = [plsc.bitcast(k - thresh, jnp.uint32) for k in kys]
                bnd = [v & (d < width_u) for v, d in zip(vls, ds_)]
                dgs = [jnp.where(b, plsc.bitcast(jnp.right_shift(d, s_u),
                                                 jnp.int32), 16)
                       for b, d in zip(bnd, ds_)]
                for d in dgs:
                    plsc.addupdate_scatter(h2_v, [l2off + d], ones)
                return jnp.int32(0)

            lax.fori_loop(0, nch8, hb_body, jnp.int32(0))

            loaded = [h2_v[pl.ds(l * 17, _LANES)] for l in range(_LANES)]
            for l in range(_LANES):
                h2_v[pl.ds(l * 17, _LANES)] = zvec
            tot = loaded[0]
            for l in range(1, _LANES):
                tot = tot + loaded[l]
            suffix = lax.rev(jnp.cumsum(lax.rev(tot, (0,))), (0,))
            nB = jnp.sum(tot).astype(jnp.int32)
            r_l = jnp.int32(_K) - (m - nB)
            b = jnp.max(jnp.where(suffix >= r_l, lanes, -1)).astype(jnp.int32)
            sufb1 = jnp.sum(jnp.where(lanes == b + 1, suffix, 0)).astype(
                jnp.int32)
            c_gt = (m - nB) + sufb1
            thresh = (thresh + jnp.left_shift(b, s)).astype(jnp.int32)

            def cb_body(c0, cnm1):
                base = c0 * (_LANES * 8)
                offs = [base + u * _LANES for u in range(8)]
                idxs = [ci_v[pl.ds(o, _LANES)] for o in offs]
                vls = [(o + lanes) < m for o in offs]
                kvs = [plsc.load_gather(cur, [i], mask=v)
                       for i, v in zip(idxs, vls)]
                kys = [_f32_key(kv) for kv in kvs]
                kp = [v & (k >= thresh) for v, k in zip(vls, kys)]
                cs = [plsc.cumsum(ones, mask=k) for k in kp]
                pc = [plsc.all_reduce_population_count(k) for k in kp]
                cns = [cnm1]
                for u in range(8):
                    cns.append(cns[-1] + pc[u])
                pos = [cns[u] + cs[u] for u in range(8)]
                for u in range(8):
                    plsc.store_scatter(ci_v, [pos[u]], idxs[u], mask=kp[u])
                return cns[8]

            cn2 = lax.fori_loop(0, nch8, cb_body, zvec - 1)
            return (thresh, (jnp.max(cn2) + 1).astype(jnp.int32), c_gt)

        thresh, m, c_gt = lax.fori_loop(
            0, 6, level_body, (thresh, m, c_gt))

        r_fin = jnp.int32(_K) - c_gt
        nchf = (m + _LANES - 1) // _LANES

        def f_body(c, carry):
            neq, nw = carry
            idx = ci_v[pl.ds(c * _LANES, _LANES)]
            valid = (c * _LANES + lanes) < m
            kv = plsc.load_gather(cur, [idx], mask=valid)
            key = _f32_key(kv)
            gt = valid & (key > thresh)
            eq = valid & (key == thresh)
            eqi = eq.astype(jnp.int32)
            tier = neq + jnp.cumsum(eqi) - eqi
            keep = gt | (eq & (tier < r_fin))
            ki = keep.astype(jnp.int32)
            pos = nw + jnp.cumsum(ki) - ki + (j * _K)
            plsc.store_scatter(wi_v, [pos], idx, mask=keep)
            return (neq + plsc.all_reduce_population_count(eq),
                    nw + plsc.all_reduce_population_count(keep))

        lax.fori_loop(0, nchf, f_body, (zvec, zvec))

        for c in range(_K // _LANES):
            iv = wi_v[pl.ds(j * _K + c * _LANES, _LANES)]
            wk_v[pl.ds(j * _K + c * _LANES, _LANES)] = plsc.load_gather(
                cur, [iv])

        if j + 1 < _ROWS_PER_W:
            pending = nxt

    pltpu.sync_copy(wk_v, outv_hbm.at[pl.ds(wid * (_ROWS_PER_W * _K),
                                            _ROWS_PER_W * _K)])
    pltpu.sync_copy(wi_v, outi_hbm.at[pl.ds(wid * (_ROWS_PER_W * _K),
                                            _ROWS_PER_W * _K)])


_sc_topk = functools.partial(
    pl.kernel,
    out_type=(
        jax.ShapeDtypeStruct((_NROWS * _K,), jnp.float32),
        jax.ShapeDtypeStruct((_NROWS * _K,), jnp.int32),
    ),
    mesh=plsc.VectorSubcoreMesh(core_axis_name="c", subcore_axis_name="s"),
    compiler_params=pltpu.CompilerParams(
        needs_layout_passes=False, use_tc_tiling_on_sc=True),
    scratch_types=[
        pltpu.VMEM((_N,), jnp.float32),
        pltpu.VMEM((_N,), jnp.float32),
        pltpu.VMEM((_N,), jnp.int32),
        pltpu.VMEM((16 * 256,), jnp.int32),
        pltpu.VMEM((16 * 17,), jnp.int32),
        pltpu.VMEM((256,), jnp.int32),
        pltpu.VMEM((_ROWS_PER_W * _K,), jnp.float32),
        pltpu.VMEM((_ROWS_PER_W * _K,), jnp.int32),
        pltpu.SemaphoreType.DMA,
        pltpu.SemaphoreType.DMA,
    ],
)(_sc_body)


def _tc_sort_body(k_ref, i_ref, vo_ref, io_ref):
    k = k_ref[...]
    ix = i_ref[...]
    lane = lax.broadcasted_iota(jnp.int32, (_NROWS, _K), 1)
    for kk in [2, 4, 8, 16, 32, 64]:
        j = kk // 2
        while j >= 1:
            lk = jnp.concatenate([k[:, j:], k[:, :j]], axis=1)
            rk = jnp.concatenate([k[:, _K - j:], k[:, :_K - j]], axis=1)
            li = jnp.concatenate([ix[:, j:], ix[:, :j]], axis=1)
            ri = jnp.concatenate([ix[:, _K - j:], ix[:, :_K - j]], axis=1)
            bitclear = (lane & j) == 0
            pk = jnp.where(bitclear, lk, rk)
            pi = jnp.where(bitclear, li, ri)
            first = (k > pk) | ((k == pk) & (ix < pi))
            forward = (lane & kk) == 0
            take = bitclear ^ first ^ (~forward)
            k = jnp.where(take, pk, k)
            ix = jnp.where(take, pi, ix)
            j //= 2
    vo_ref[...] = k
    io_ref[...] = ix


_tc_sort = pl.pallas_call(
    _tc_sort_body,
    out_shape=(
        jax.ShapeDtypeStruct((_NROWS, _K), jnp.float32),
        jax.ShapeDtypeStruct((_NROWS, _K), jnp.int32),
    ),
)


def kernel(x):
    wv, wi = _sc_topk(x)
    return _tc_sort(wv.reshape(_NROWS, _K), wi.reshape(_NROWS, _K))
